# baseline (speedup 1.0000x reference)
import functools

import jax
import jax.numpy as jnp
from jax import lax
from jax.experimental import pallas as pl
from jax.experimental.pallas import tpu as pltpu
from jax.experimental.pallas import tpu_sc as plsc

NC = 2
NS = 16
NW = NC * NS
LANES = 16
CHUNK = 128


def _build(B, H):
    bpw = B // NW
    nch = bpw // CHUNK
    ngrp = bpw // LANES

    mesh = plsc.VectorSubcoreMesh(core_axis_name="c", subcore_axis_name="s")

    @functools.partial(
        pl.kernel,
        out_type=jax.ShapeDtypeStruct((B,), jnp.float32),
        mesh=mesh,
        compiler_params=pltpu.CompilerParams(
            needs_layout_passes=False, use_tc_tiling_on_sc=False),
        scratch_types=[
            pltpu.VMEM((nch, CHUNK), jnp.int32),
            pltpu.VMEM((nch, CHUNK), jnp.int32),
            pltpu.VMEM((bpw, H), jnp.float32),
            pltpu.VMEM((bpw, H), jnp.float32),
            pltpu.VMEM((bpw,), jnp.float32),
            pltpu.VMEM((bpw,), jnp.float32),
            pltpu.VMEM((bpw,), jnp.float32),
            pltpu.VMEM((LANES,), jnp.float32),
            pltpu.SemaphoreType.DMA,
        ],
    )
    def mf(user_hbm, item_hbm, uw_hbm, iw_hbm, ub_hbm, ib_hbm, bias_hbm,
           out_hbm, uidx_v, iidx_v, urows_v, irows_v, ubr_v, ibr_v,
           out_v, bias_v, sem):
        wid = lax.axis_index("s") * NC + lax.axis_index("c")
        base = wid * bpw

        pltpu.sync_copy(user_hbm.at[wid], uidx_v)
        pltpu.sync_copy(item_hbm.at[wid], iidx_v)
        pltpu.sync_copy(bias_hbm, bias_v)

        copies = []
        for c in range(nch):
            sl = pl.ds(c * CHUNK, CHUNK)
            copies.append(pltpu.make_async_copy(
                uw_hbm.at[uidx_v.at[c]], urows_v.at[sl], sem))
            copies.append(pltpu.make_async_copy(
                iw_hbm.at[iidx_v.at[c]], irows_v.at[sl], sem))
            copies.append(pltpu.make_async_copy(
                ub_hbm.at[uidx_v.at[c]], ubr_v.at[sl], sem))
            copies.append(pltpu.make_async_copy(
                ib_hbm.at[iidx_v.at[c]], ibr_v.at[sl], sem))
        for cp in copies:
            cp.start()
        for cp in copies:
            cp.wait()

        bias_vec = bias_v[...]
        iot = lax.iota(jnp.int32, LANES)

        def group_body(g, carry):
            rowidx = g * LANES + iot
            ubv = ubr_v[pl.ds(g * LANES, LANES)]
            ibv = ibr_v[pl.ds(g * LANES, LANES)]

            def h_body(h, acc):
                colidx = jnp.full((LANES,), h, jnp.int32)
                cu = plsc.load_gather(urows_v, [rowidx, colidx])
                ci = plsc.load_gather(irows_v, [rowidx, colidx])
                return acc + (cu + ubv) * (ci + ibv)

            acc = lax.fori_loop(0, H, h_body, jnp.zeros((LANES,), jnp.float32))
            out_v[pl.ds(g * LANES, LANES)] = acc + bias_vec
            return carry

        lax.fori_loop(0, ngrp, group_body, 0)
        pltpu.sync_copy(out_v, out_hbm.at[pl.ds(base, bpw)])

    return mf


def kernel(user, item, user_weight, item_weight, user_bias, item_bias, bias):
    B = user.shape[0]
    H = user_weight.shape[1]
    user_r = user.reshape(NW, B // NW // CHUNK, CHUNK)
    item_r = item.reshape(NW, B // NW // CHUNK, CHUNK)
    ub = user_bias.reshape(-1)
    ib = item_bias.reshape(-1)
    bias8 = jnp.broadcast_to(bias, (LANES,)).astype(jnp.float32)
    mf = _build(B, H)
    return mf(user_r, item_r, user_weight, item_weight, ub, ib, bias8)
